# baseline (speedup 1.0000x reference)
import functools

import jax
import jax.numpy as jnp
from jax import lax
from jax.experimental import pallas as pl
from jax.experimental.pallas import tpu as pltpu
from jax.experimental.pallas import tpu_sc as plsc

B = 16384
D = 128
NAGE = 120
NGND = 4
NCOMBO = NAGE * NGND

NC = 2
NS = 16
NW = NC * NS
BPW = B // NW
CH = 64
NCH = BPW // CH
LANES = 16


def _combo_table_body(age_t_ref, gnd_t_ref, gamma_ref, beta_ref, t_ref):
    at = age_t_ref[...]
    gt = gnd_t_ref[...]
    s = (jnp.sum(at, axis=1, keepdims=True)[None, :, :]
         + jnp.sum(gt, axis=1, keepdims=True)[:, None, :])
    mean = s / D
    ca = at[None, :, :] - mean
    cg = gt[:, None, :] - mean
    var = (jnp.sum(ca * ca, axis=2, keepdims=True)
           + jnp.sum(cg * cg, axis=2, keepdims=True)) / D
    rstd = lax.rsqrt(var + 1e-6)
    gamma = gamma_ref[...]
    beta = beta_ref[...]
    left = ca * rstd * gamma[None, :, :64] + beta[None, :, :64]
    right = cg * rstd * gamma[None, :, 64:] + beta[None, :, 64:]
    t_ref[...] = jnp.concatenate([left, right], axis=-1)


def _build_combo_table(age_table, gnd_table, gamma, beta):
    t3 = pl.pallas_call(
        _combo_table_body,
        out_shape=jax.ShapeDtypeStruct((NGND, NAGE, D), jnp.float32),
    )(age_table, gnd_table, gamma.reshape(1, D), beta.reshape(1, D))
    return t3.reshape(NCOMBO, D)


def _sc_gather_body(age_hbm, gnd_hbm, t_hbm, out_hbm,
                    cidx, av, gv, rows, tspm,
                    g0, g1, g2, g3, g4, g5, g6, g7, ia, ig, it, ssem):
    gsems = (g0, g1, g2, g3, g4, g5, g6, g7)
    sid = lax.axis_index("s")
    wid = sid * NC + lax.axis_index("c")
    base = wid * BPW
    age_cp = pltpu.async_copy(age_hbm.at[pl.ds(base, BPW)], av, ia)
    gnd_cp = pltpu.async_copy(gnd_hbm.at[pl.ds(base, BPW)], gv, ig)
    t_cp = None
    @pl.when(sid == 0)
    def _():
        nonlocal t_cp
        t_cp = pltpu.async_copy(t_hbm, tspm, it)
    age_cp.wait()
    gnd_cp.wait()
    gathers = []
    for k in range(NCH):
        for i in range(CH // LANES):
            sl = pl.ds(i * LANES, LANES)
            src = pl.ds(k * CH + i * LANES, LANES)
            cidx[k, sl] = gv[src] * NAGE + av[src]
        if k == 0:
            @pl.when(sid == 0)
            def _():
                t_cp.wait()
            plsc.subcore_barrier()
        gathers.append(
            pltpu.async_copy(tspm.at[cidx.at[k]], rows.at[k], gsems[k])
        )
    scatters = []
    for k in range(NCH):
        gathers[k].wait()
        scatters.append(
            pltpu.async_copy(rows.at[k], out_hbm.at[pl.ds(base + k * CH, CH)], ssem)
        )
    for s in scatters:
        s.wait()


@functools.lru_cache(maxsize=None)
def _make_sc_gather():
    mesh = plsc.VectorSubcoreMesh(
        core_axis_name="c", subcore_axis_name="s", num_cores=NC, num_subcores=NS
    )
    return pl.kernel(
        _sc_gather_body,
        out_type=jax.ShapeDtypeStruct((B, D), jnp.float32),
        mesh=mesh,
        scratch_types=[
            pltpu.VMEM((NCH, CH), jnp.int32),
            pltpu.VMEM((BPW,), jnp.int32),
            pltpu.VMEM((BPW,), jnp.int32),
            pltpu.VMEM((NCH, CH, D), jnp.float32),
            pltpu.VMEM_SHARED((NCOMBO, D), jnp.float32),
            pltpu.SemaphoreType.DMA,
            pltpu.SemaphoreType.DMA,
            pltpu.SemaphoreType.DMA,
            pltpu.SemaphoreType.DMA,
            pltpu.SemaphoreType.DMA,
            pltpu.SemaphoreType.DMA,
            pltpu.SemaphoreType.DMA,
            pltpu.SemaphoreType.DMA,
            pltpu.SemaphoreType.DMA,
            pltpu.SemaphoreType.DMA,
            pltpu.SemaphoreType.DMA,
            pltpu.SemaphoreType.DMA,
        ],
    )


def _tc_tail_body(age_ref, gnd_ref, t_ref, sc_ref, out_ref):
    del sc_ref
    c = gnd_ref[...] * NAGE + age_ref[...]
    oh = (c == lax.broadcasted_iota(jnp.int32, (TCROWS, NCOMBO), 1))
    out_ref[...] = jnp.dot(oh.astype(jnp.float32), t_ref[...],
                           preferred_element_type=jnp.float32,
                           precision=lax.Precision.HIGHEST)


TCROWS = 512


def kernel(age, gnd, age_table, gnd_table, gamma, beta):
    age = age.astype(jnp.int32)
    gnd = gnd.astype(jnp.int32)
    t = _build_combo_table(age_table, gnd_table, gamma, beta)
    sc_out = _make_sc_gather()(age, gnd, t)
    lo = B - TCROWS
    out = pl.pallas_call(
        _tc_tail_body,
        out_shape=jax.ShapeDtypeStruct((B, D), jnp.float32),
        grid=(1,),
        in_specs=[
            pl.BlockSpec((TCROWS, 1), lambda i: (lo // TCROWS, 0)),
            pl.BlockSpec((TCROWS, 1), lambda i: (lo // TCROWS, 0)),
            pl.BlockSpec((NCOMBO, D), lambda i: (0, 0)),
            pl.BlockSpec((TCROWS, D), lambda i: (lo // TCROWS, 0)),
        ],
        out_specs=pl.BlockSpec((TCROWS, D), lambda i: (lo // TCROWS, 0)),
        input_output_aliases={3: 0},
    )(age.reshape(B, 1), gnd.reshape(B, 1), t, sc_out)
    return out


# speedup vs baseline: 1.3418x; 1.3418x over previous
import functools

import jax
import jax.numpy as jnp
from jax import lax
from jax.experimental import pallas as pl
from jax.experimental.pallas import tpu as pltpu
from jax.experimental.pallas import tpu_sc as plsc

B = 16384
D = 128
NAGE = 120
NGND = 4
NCOMBO = NAGE * NGND

NC = 2
NS = 16
NW = NC * NS
BPW = B // NW
CH = 64
NCH = BPW // CH
LANES = 16


def _combo_table_body(age_t_ref, gnd_t_ref, gamma_ref, beta_ref, t_ref):
    at = age_t_ref[...]
    gt = gnd_t_ref[...]
    s = (jnp.sum(at, axis=1, keepdims=True)[None, :, :]
         + jnp.sum(gt, axis=1, keepdims=True)[:, None, :])
    mean = s / D
    ca = at[None, :, :] - mean
    cg = gt[:, None, :] - mean
    var = (jnp.sum(ca * ca, axis=2, keepdims=True)
           + jnp.sum(cg * cg, axis=2, keepdims=True)) / D
    rstd = lax.rsqrt(var + 1e-6)
    gamma = gamma_ref[...]
    beta = beta_ref[...]
    left = ca * rstd * gamma[None, :, :64] + beta[None, :, :64]
    right = cg * rstd * gamma[None, :, 64:] + beta[None, :, 64:]
    t_ref[...] = jnp.concatenate([left, right], axis=-1)


def _build_combo_table(age_table, gnd_table, gamma, beta):
    t3 = pl.pallas_call(
        _combo_table_body,
        out_shape=jax.ShapeDtypeStruct((NGND, NAGE, D), jnp.float32),
    )(age_table, gnd_table, gamma.reshape(1, D), beta.reshape(1, D))
    return t3.reshape(NCOMBO, D)


def _sc_gather_body(age_hbm, gnd_hbm, t_hbm, out_hbm,
                    cidx, av, gv, rows, tspm,
                    g0, g1, g2, g3, g4, g5, g6, g7, ia, ig, it, ssem):
    gsems = (g0, g1, g2, g3, g4, g5, g6, g7)
    sid = lax.axis_index("s")
    wid = sid * NC + lax.axis_index("c")
    base = wid * BPW
    age_cp = pltpu.async_copy(age_hbm.at[pl.ds(base, BPW)], av, ia)
    gnd_cp = pltpu.async_copy(gnd_hbm.at[pl.ds(base, BPW)], gv, ig)
    t_cp = None
    @pl.when(sid == 0)
    def _():
        nonlocal t_cp
        t_cp = pltpu.async_copy(t_hbm, tspm, it)
    age_cp.wait()
    gnd_cp.wait()
    gathers = []
    for k in range(NCH):
        for i in range(CH // LANES):
            sl = pl.ds(i * LANES, LANES)
            src = pl.ds(k * CH + i * LANES, LANES)
            cidx[k, sl] = gv[src] * NAGE + av[src]
        if k == 0:
            @pl.when(sid == 0)
            def _():
                t_cp.wait()
            plsc.subcore_barrier()
        gathers.append(
            pltpu.async_copy(tspm.at[cidx.at[k]], rows.at[k], gsems[k])
        )
    scatters = []
    for k in range(NCH):
        gathers[k].wait()
        scatters.append(
            pltpu.async_copy(rows.at[k], out_hbm.at[pl.ds(base + k * CH, CH)], ssem)
        )
    for s in scatters:
        s.wait()


@functools.lru_cache(maxsize=None)
def _make_sc_gather():
    mesh = plsc.VectorSubcoreMesh(
        core_axis_name="c", subcore_axis_name="s", num_cores=NC, num_subcores=NS
    )
    return pl.kernel(
        _sc_gather_body,
        out_type=jax.ShapeDtypeStruct((B, D), jnp.float32),
        mesh=mesh,
        scratch_types=[
            pltpu.VMEM((NCH, CH), jnp.int32),
            pltpu.VMEM((BPW,), jnp.int32),
            pltpu.VMEM((BPW,), jnp.int32),
            pltpu.VMEM((NCH, CH, D), jnp.float32),
            pltpu.VMEM_SHARED((NCOMBO, D), jnp.float32),
            pltpu.SemaphoreType.DMA,
            pltpu.SemaphoreType.DMA,
            pltpu.SemaphoreType.DMA,
            pltpu.SemaphoreType.DMA,
            pltpu.SemaphoreType.DMA,
            pltpu.SemaphoreType.DMA,
            pltpu.SemaphoreType.DMA,
            pltpu.SemaphoreType.DMA,
            pltpu.SemaphoreType.DMA,
            pltpu.SemaphoreType.DMA,
            pltpu.SemaphoreType.DMA,
            pltpu.SemaphoreType.DMA,
        ],
    )


def kernel(age, gnd, age_table, gnd_table, gamma, beta):
    age = age.astype(jnp.int32)
    gnd = gnd.astype(jnp.int32)
    t = _build_combo_table(age_table, gnd_table, gamma, beta)
    return _make_sc_gather()(age, gnd, t)
